# baseline (speedup 1.0000x reference)
import functools
import jax
import jax.numpy as jnp
from jax import lax
from jax.experimental import pallas as pl
from jax.experimental.pallas import tpu as pltpu
from jax.experimental.pallas import tpu_sc as plsc

N = 10000
E = 320000
D = 128
DE = 16
HID = 512
G = 256
DEPTH = 3

NPAD = 10240

NTILE = 16
NCORE = 2
RANGE = NPAD // 64
TRASH = RANGE
SEG_V = 125
SELCAP = SEG_V * 16
NSEG = E // SELCAP
CHUNK = 32
NCH = HID // 16


def _pr_body(x_ref, w_ref, b_ref, p_ref, r_ref):
    y = jnp.dot(x_ref[...], w_ref[...], preferred_element_type=jnp.float32)
    p = jnp.exp(y + b_ref[...][None, :])
    p_ref[...] = p
    r_ref[...] = p / jnp.sum(p, axis=1, keepdims=True)


def _node_pr(x, Wx, b):
    din = x.shape[1]
    blk = 256
    grid = NPAD // blk
    return pl.pallas_call(
        _pr_body,
        grid=(grid,),
        in_specs=[
            pl.BlockSpec((blk, din), lambda i: (i, 0)),
            pl.BlockSpec((din, HID), lambda i: (0, 0)),
            pl.BlockSpec((HID,), lambda i: (0,)),
        ],
        out_specs=[
            pl.BlockSpec((blk, HID), lambda i: (i, 0)),
            pl.BlockSpec((blk, HID), lambda i: (i, 0)),
        ],
        out_shape=[
            jax.ShapeDtypeStruct((NPAD, HID), jnp.float32),
            jax.ShapeDtypeStruct((NPAD, HID), jnp.float32),
        ],
    )(x, Wx, b)


def _q_body(ea_ref, w_ref, q_ref):
    q_ref[...] = jnp.exp(
        jnp.dot(ea_ref[...], w_ref[...], preferred_element_type=jnp.float32))


def _edge_q(edge_attr, We):
    blk = 1280
    grid = E // blk
    return pl.pallas_call(
        _q_body,
        grid=(grid,),
        in_specs=[
            pl.BlockSpec((blk, DE), lambda i: (i, 0)),
            pl.BlockSpec((DE, HID), lambda i: (0, 0)),
        ],
        out_specs=pl.BlockSpec((blk, HID), lambda i: (i, 0)),
        out_shape=jax.ShapeDtypeStruct((E, HID), jnp.float32),
    )(edge_attr, We)


def _msg_body(pg_ref, q_ref, m_ref):
    u = pg_ref[...] * q_ref[...]
    m_ref[...] = u / jnp.sum(u, axis=1, keepdims=True)


def _edge_msg(Pg, Q):
    blk = 1280
    grid = E // blk
    return pl.pallas_call(
        _msg_body,
        grid=(grid,),
        in_specs=[
            pl.BlockSpec((blk, HID), lambda i: (i, 0)),
            pl.BlockSpec((blk, HID), lambda i: (i, 0)),
        ],
        out_specs=pl.BlockSpec((blk, HID), lambda i: (i, 0)),
        out_shape=jax.ShapeDtypeStruct((E, HID), jnp.float32),
    )(Pg, Q)


def _pool_body(b_ref, x_ref, acc_ref):
    i = pl.program_id(0)
    seg = b_ref[...]
    onehot = (seg == jax.lax.broadcasted_iota(jnp.int32, (G, seg.shape[1]), 0)
              ).astype(jnp.float32)
    part = jnp.dot(onehot, x_ref[...], preferred_element_type=jnp.float32)

    @pl.when(i == 0)
    def _init():
        acc_ref[...] = part

    @pl.when(i > 0)
    def _acc():
        acc_ref[...] += part


def _pool(batch_padded, x):
    blk = 1024
    grid = NPAD // blk
    return pl.pallas_call(
        _pool_body,
        grid=(grid,),
        in_specs=[
            pl.BlockSpec((1, blk), lambda i: (0, i)),
            pl.BlockSpec((blk, HID), lambda i: (i, 0)),
        ],
        out_specs=pl.BlockSpec((G, HID), lambda i: (0, 0)),
        out_shape=jax.ShapeDtypeStruct((G, HID), jnp.float32),
    )(batch_padded.reshape(1, NPAD), x)


def _readout_body(p_ref, w3_ref, b3_ref, w4_ref, b4_ref, o_ref):
    h = jnp.dot(p_ref[...], w3_ref[...], preferred_element_type=jnp.float32)
    h = h + b3_ref[...][None, :]
    o = jnp.dot(h, w4_ref[...], preferred_element_type=jnp.float32)
    o_ref[...] = o + b4_ref[...][None, :]


def _readout(pooled, W3, b3, W4, b4):
    return pl.pallas_call(
        _readout_body,
        in_specs=[pl.BlockSpec(pooled.shape, lambda: (0, 0)),
                  pl.BlockSpec(W3.shape, lambda: (0, 0)),
                  pl.BlockSpec(b3.shape, lambda: (0,)),
                  pl.BlockSpec(W4.shape, lambda: (0, 0)),
                  pl.BlockSpec(b4.shape, lambda: (0,))],
        out_specs=pl.BlockSpec((G, 1), lambda: (0, 0)),
        out_shape=jax.ShapeDtypeStruct((G, 1), jnp.float32),
    )(pooled, W3, b3, W4, b4)


def _sc_conv_body(p_hbm, q_hbm, r_hbm, src_hbm, dst_hbm, out_hbm,
                  dstspan, srcspan, sel_src, sel_eid, sel_ldst,
                  csrc, ceid, cldst, prows, qrows, accum, sem1, sem2):
    cid = lax.axis_index("c")
    tid = lax.axis_index("s")
    wid = cid * NTILE + tid
    iota16 = lax.iota(jnp.int32, 16)

    def process_chunk(j, _):
        off = j * CHUNK
        for v in range(CHUNK // 16):
            csrc[pl.ds(v * 16, 16)] = sel_src[pl.ds(off + v * 16, 16)]
            ceid[pl.ds(v * 16, 16)] = sel_eid[pl.ds(off + v * 16, 16)]
            cldst[pl.ds(v * 16, 16)] = sel_ldst[pl.ds(off + v * 16, 16)]
        h1 = pltpu.async_copy(p_hbm.at[csrc], prows, sem1)
        h2 = pltpu.async_copy(q_hbm.at[ceid], qrows, sem2)
        h1.wait()
        h2.wait()

        def edge_body(e, _):
            ldst = cldst[pl.ds(e, 16)][0]
            ms = []
            sacc = None
            for c in range(NCH):
                p = prows[e, pl.ds(c * 16, 16)]
                q = qrows[e, pl.ds(c * 16, 16)]
                m = p * q
                ms.append(m)
                sacc = m if sacc is None else sacc + m
            inv16 = 1.0 / jnp.full((16,), jnp.sum(sacc), jnp.float32)
            for c in range(NCH):
                sl = pl.ds(c * 16, 16)
                accum[ldst, sl] = accum[ldst, sl] + ms[c] * inv16
            return 0

        lax.fori_loop(0, CHUNK, edge_body, 0)
        return 0

    for r in range(2):
        base = (r * NTILE * NCORE + wid) * RANGE
        pltpu.sync_copy(r_hbm.at[pl.ds(base, RANGE)],
                        accum.at[pl.ds(0, RANGE)])

        def seg_body(seg, _):
            seg_lo = pl.multiple_of(seg * SELCAP, 8)
            pltpu.sync_copy(dst_hbm.at[pl.ds(seg_lo, SELCAP)], dstspan)
            pltpu.sync_copy(src_hbm.at[pl.ds(seg_lo, SELCAP)], srcspan)

            def scan_body(v, cnt):
                pos = v * 16
                ld_vec = dstspan[pl.ds(pos, 16)] - base
                s_vec = srcspan[pl.ds(pos, 16)]
                e_vec = seg_lo + pos + iota16
                msk = (ld_vec >= 0) & (ld_vec < RANGE)
                mi = msk.astype(jnp.int32)
                tgt = cnt + plsc.cumsum(mi) - mi
                plsc.store_scatter(sel_src, [tgt], s_vec, mask=msk)
                plsc.store_scatter(sel_eid, [tgt], e_vec, mask=msk)
                plsc.store_scatter(sel_ldst, [tgt], ld_vec, mask=msk)
                return cnt + jnp.sum(mi)

            nsel = lax.fori_loop(0, SEG_V, scan_body, jnp.int32(0))
            nchunks = (nsel + CHUNK - 1) // CHUNK
            npad_to = nchunks * CHUNK
            vbase = (nsel // 16) * 16
            for v in range(1 + (CHUNK + 15) // 16):
                offp = vbase + v * 16

                @pl.when(offp < npad_to)
                def _pad():
                    posv = offp + iota16
                    keep = posv < nsel
                    sel_src[pl.ds(offp, 16)] = jnp.where(
                        keep, sel_src[pl.ds(offp, 16)], 0)
                    sel_eid[pl.ds(offp, 16)] = jnp.where(
                        keep, sel_eid[pl.ds(offp, 16)], 0)
                    sel_ldst[pl.ds(offp, 16)] = jnp.where(
                        keep, sel_ldst[pl.ds(offp, 16)], TRASH)

            lax.fori_loop(0, nchunks, process_chunk, 0)
            return 0

        lax.fori_loop(0, NSEG, seg_body, 0)
        pltpu.sync_copy(accum.at[pl.ds(0, RANGE)],
                        out_hbm.at[pl.ds(base, RANGE)])


def _sc_conv(P, Q, R, src, dst):
    mesh = plsc.VectorSubcoreMesh(core_axis_name="c", subcore_axis_name="s")
    f = pl.kernel(
        _sc_conv_body,
        mesh=mesh,
        compiler_params=pltpu.CompilerParams(needs_layout_passes=False),
        out_type=jax.ShapeDtypeStruct((NPAD, HID), jnp.float32),
        scratch_types=[
            pltpu.VMEM((SELCAP,), jnp.int32),
            pltpu.VMEM((SELCAP,), jnp.int32),
            pltpu.VMEM((SELCAP + CHUNK + 16,), jnp.int32),
            pltpu.VMEM((SELCAP + CHUNK + 16,), jnp.int32),
            pltpu.VMEM((SELCAP + CHUNK + 16,), jnp.int32),
            pltpu.VMEM((CHUNK,), jnp.int32),
            pltpu.VMEM((CHUNK,), jnp.int32),
            pltpu.VMEM((CHUNK + 16,), jnp.int32),
            pltpu.VMEM((CHUNK, HID), jnp.float32),
            pltpu.VMEM((CHUNK, HID), jnp.float32),
            pltpu.VMEM((RANGE + 1, HID), jnp.float32),
            pltpu.SemaphoreType.DMA,
            pltpu.SemaphoreType.DMA,
        ],
    )
    return f(P, Q, R, src, dst)


def kernel(x, edge_index, edge_attr, smiles, batch, W1, b1, W2, b2, W3, b3, W4, b4):
    src, dst = edge_index[0], edge_index[1]
    Q1 = _edge_q(edge_attr, W1[D:])
    Q2 = _edge_q(edge_attr, W2[HID:])

    h = jnp.zeros((NPAD, D), jnp.float32).at[:N].set(x)
    for layer in range(1 + DEPTH):
        if layer == 0:
            P, R = _node_pr(h, W1[:D], b1)
            Q = Q1
        else:
            P, R = _node_pr(h, W2[:HID], b2)
            Q = Q2
        h = _sc_conv(P, Q, R, src, dst)

    batch_padded = jnp.full((NPAD,), G, jnp.int32).at[:N].set(batch)
    pooled = _pool(batch_padded, h)
    return _readout(pooled, W3, b3, W4, b4)


# speedup vs baseline: 1.7252x; 1.7252x over previous
import functools
import jax
import jax.numpy as jnp
from jax import lax
from jax.experimental import pallas as pl
from jax.experimental.pallas import tpu as pltpu
from jax.experimental.pallas import tpu_sc as plsc

N = 10000
E = 320000
D = 128
DE = 16
HID = 512
G = 256
DEPTH = 3

NPAD = 10240

NTILE = 16
NCORE = 2
RANGE = NPAD // 64
TRASH = RANGE
SEG_V = 125
SELCAP = SEG_V * 16
NSEG = E // SELCAP
CHUNK = 16
NCH = HID // 16


def _pr_body(x_ref, w_ref, b_ref, p_ref, r_ref):
    y = jnp.dot(x_ref[...], w_ref[...], preferred_element_type=jnp.float32)
    p = jnp.exp(y + b_ref[...][None, :])
    p_ref[...] = p
    r_ref[...] = p / jnp.sum(p, axis=1, keepdims=True)


def _node_pr(x, Wx, b):
    din = x.shape[1]
    blk = 256
    grid = NPAD // blk
    return pl.pallas_call(
        _pr_body,
        grid=(grid,),
        in_specs=[
            pl.BlockSpec((blk, din), lambda i: (i, 0)),
            pl.BlockSpec((din, HID), lambda i: (0, 0)),
            pl.BlockSpec((HID,), lambda i: (0,)),
        ],
        out_specs=[
            pl.BlockSpec((blk, HID), lambda i: (i, 0)),
            pl.BlockSpec((blk, HID), lambda i: (i, 0)),
        ],
        out_shape=[
            jax.ShapeDtypeStruct((NPAD, HID), jnp.float32),
            jax.ShapeDtypeStruct((NPAD, HID), jnp.float32),
        ],
    )(x, Wx, b)


def _q_body(ea_ref, w_ref, q_ref):
    q_ref[...] = jnp.exp(
        jnp.dot(ea_ref[...], w_ref[...], preferred_element_type=jnp.float32))


def _edge_q(edge_attr, We):
    blk = 1280
    grid = E // blk
    return pl.pallas_call(
        _q_body,
        grid=(grid,),
        in_specs=[
            pl.BlockSpec((blk, DE), lambda i: (i, 0)),
            pl.BlockSpec((DE, HID), lambda i: (0, 0)),
        ],
        out_specs=pl.BlockSpec((blk, HID), lambda i: (i, 0)),
        out_shape=jax.ShapeDtypeStruct((E, HID), jnp.float32),
    )(edge_attr, We)


def _msg_body(pg_ref, q_ref, m_ref):
    u = pg_ref[...] * q_ref[...]
    m_ref[...] = u / jnp.sum(u, axis=1, keepdims=True)


def _edge_msg(Pg, Q):
    blk = 1280
    grid = E // blk
    return pl.pallas_call(
        _msg_body,
        grid=(grid,),
        in_specs=[
            pl.BlockSpec((blk, HID), lambda i: (i, 0)),
            pl.BlockSpec((blk, HID), lambda i: (i, 0)),
        ],
        out_specs=pl.BlockSpec((blk, HID), lambda i: (i, 0)),
        out_shape=jax.ShapeDtypeStruct((E, HID), jnp.float32),
    )(Pg, Q)


def _pool_body(b_ref, x_ref, acc_ref):
    i = pl.program_id(0)
    seg = b_ref[...]
    onehot = (seg == jax.lax.broadcasted_iota(jnp.int32, (G, seg.shape[1]), 0)
              ).astype(jnp.float32)
    part = jnp.dot(onehot, x_ref[...], preferred_element_type=jnp.float32)

    @pl.when(i == 0)
    def _init():
        acc_ref[...] = part

    @pl.when(i > 0)
    def _acc():
        acc_ref[...] += part


def _pool(batch_padded, x):
    blk = 1024
    grid = NPAD // blk
    return pl.pallas_call(
        _pool_body,
        grid=(grid,),
        in_specs=[
            pl.BlockSpec((1, blk), lambda i: (0, i)),
            pl.BlockSpec((blk, HID), lambda i: (i, 0)),
        ],
        out_specs=pl.BlockSpec((G, HID), lambda i: (0, 0)),
        out_shape=jax.ShapeDtypeStruct((G, HID), jnp.float32),
    )(batch_padded.reshape(1, NPAD), x)


def _readout_body(p_ref, w3_ref, b3_ref, w4_ref, b4_ref, o_ref):
    h = jnp.dot(p_ref[...], w3_ref[...], preferred_element_type=jnp.float32)
    h = h + b3_ref[...][None, :]
    o = jnp.dot(h, w4_ref[...], preferred_element_type=jnp.float32)
    o_ref[...] = o + b4_ref[...][None, :]


def _readout(pooled, W3, b3, W4, b4):
    return pl.pallas_call(
        _readout_body,
        in_specs=[pl.BlockSpec(pooled.shape, lambda: (0, 0)),
                  pl.BlockSpec(W3.shape, lambda: (0, 0)),
                  pl.BlockSpec(b3.shape, lambda: (0,)),
                  pl.BlockSpec(W4.shape, lambda: (0, 0)),
                  pl.BlockSpec(b4.shape, lambda: (0,))],
        out_specs=pl.BlockSpec((G, 1), lambda: (0, 0)),
        out_shape=jax.ShapeDtypeStruct((G, 1), jnp.float32),
    )(pooled, W3, b3, W4, b4)


def _sc_conv_body(p_hbm, q_hbm, r_hbm, src_hbm, dst_hbm, out_hbm,
                  dstspan, srcspan, sel_src, sel_eid, sel_ldst,
                  csrc0, ceid0, cldst0, prows0, qrows0,
                  csrc1, ceid1, cldst1, prows1, qrows1,
                  accum, semp0, semq0, semp1, semq1):
    cid = lax.axis_index("c")
    tid = lax.axis_index("s")
    wid = cid * NTILE + tid
    iota16 = lax.iota(jnp.int32, 16)
    bufs = ((csrc0, ceid0, cldst0, prows0, qrows0, semp0, semq0),
            (csrc1, ceid1, cldst1, prows1, qrows1, semp1, semq1))

    def issue(par, j):
        csrc, ceid, cldst, prows, qrows, semp, semq = bufs[par]
        off = j * CHUNK
        csrc[pl.ds(0, 16)] = sel_src[pl.ds(off, 16)]
        ceid[pl.ds(0, 16)] = sel_eid[pl.ds(off, 16)]
        cldst[pl.ds(0, 16)] = sel_ldst[pl.ds(off, 16)]
        pltpu.make_async_copy(p_hbm.at[csrc], prows, semp).start()
        pltpu.make_async_copy(q_hbm.at[ceid], qrows, semq).start()

    def consume(par):
        csrc, ceid, cldst, prows, qrows, semp, semq = bufs[par]
        pltpu.make_async_copy(p_hbm.at[csrc], prows, semp).wait()
        pltpu.make_async_copy(q_hbm.at[ceid], qrows, semq).wait()

        def edge_body(e, _):
            ldst = cldst[pl.ds(e, 16)][0]
            ms = []
            sacc = None
            for c in range(NCH):
                p = prows[e, pl.ds(c * 16, 16)]
                q = qrows[e, pl.ds(c * 16, 16)]
                m = p * q
                ms.append(m)
                sacc = m if sacc is None else sacc + m
            inv16 = 1.0 / jnp.full((16,), jnp.sum(sacc), jnp.float32)
            for c in range(NCH):
                sl = pl.ds(c * 16, 16)
                accum[ldst, sl] = accum[ldst, sl] + ms[c] * inv16
            return 0

        lax.fori_loop(0, CHUNK, edge_body, 0)

    for r in range(2):
        base = (r * NTILE * NCORE + wid) * RANGE
        pltpu.sync_copy(r_hbm.at[pl.ds(base, RANGE)],
                        accum.at[pl.ds(0, RANGE)])

        def seg_body(seg, _):
            seg_lo = pl.multiple_of(seg * SELCAP, 8)
            pltpu.sync_copy(dst_hbm.at[pl.ds(seg_lo, SELCAP)], dstspan)
            pltpu.sync_copy(src_hbm.at[pl.ds(seg_lo, SELCAP)], srcspan)

            def scan_body(v, cnt_vec):
                pos = v * 16
                ld_vec = dstspan[pl.ds(pos, 16)] - base
                s_vec = srcspan[pl.ds(pos, 16)]
                e_vec = seg_lo + pos + iota16
                msk = plsc.bitcast(ld_vec, jnp.uint32) < jnp.uint32(RANGE)
                mi = msk.astype(jnp.int32)
                tgt = cnt_vec + plsc.cumsum(mi) - mi
                plsc.store_scatter(sel_src, [tgt], s_vec, mask=msk)
                plsc.store_scatter(sel_eid, [tgt], e_vec, mask=msk)
                plsc.store_scatter(sel_ldst, [tgt], ld_vec, mask=msk)
                return cnt_vec + plsc.all_reduce_population_count(msk)

            cnt_vec = lax.fori_loop(0, SEG_V, scan_body,
                                    jnp.zeros((16,), jnp.int32))
            nsel = cnt_vec[0]
            nchunks = (nsel + CHUNK - 1) // CHUNK
            npad_to = nchunks * CHUNK
            vbase = (nsel // 16) * 16
            for v in range(1 + CHUNK // 16):
                offp = vbase + v * 16

                @pl.when(offp < npad_to)
                def _pad():
                    posv = offp + iota16
                    keep = posv < nsel
                    sel_src[pl.ds(offp, 16)] = jnp.where(
                        keep, sel_src[pl.ds(offp, 16)], 0)
                    sel_eid[pl.ds(offp, 16)] = jnp.where(
                        keep, sel_eid[pl.ds(offp, 16)], 0)
                    sel_ldst[pl.ds(offp, 16)] = jnp.where(
                        keep, sel_ldst[pl.ds(offp, 16)], TRASH)

            @pl.when(nchunks > 0)
            def _prime():
                issue(0, 0)

            def chunk_loop(j, _):
                par = j % 2

                @pl.when(j + 1 < nchunks)
                def _next():
                    @pl.when(par == 0)
                    def _n1():
                        issue(1, j + 1)

                    @pl.when(par == 1)
                    def _n0():
                        issue(0, j + 1)

                @pl.when(par == 0)
                def _c0():
                    consume(0)

                @pl.when(par == 1)
                def _c1():
                    consume(1)

                return 0

            lax.fori_loop(0, nchunks, chunk_loop, 0)
            return 0

        lax.fori_loop(0, NSEG, seg_body, 0)
        pltpu.sync_copy(accum.at[pl.ds(0, RANGE)],
                        out_hbm.at[pl.ds(base, RANGE)])


def _sc_conv(P, Q, R, src, dst):
    mesh = plsc.VectorSubcoreMesh(core_axis_name="c", subcore_axis_name="s")
    f = pl.kernel(
        _sc_conv_body,
        mesh=mesh,
        compiler_params=pltpu.CompilerParams(needs_layout_passes=False),
        out_type=jax.ShapeDtypeStruct((NPAD, HID), jnp.float32),
        scratch_types=(
            [pltpu.VMEM((SELCAP,), jnp.int32)] * 2
            + [pltpu.VMEM((SELCAP + CHUNK + 16,), jnp.int32)] * 3
            + [pltpu.VMEM((CHUNK,), jnp.int32),
               pltpu.VMEM((CHUNK,), jnp.int32),
               pltpu.VMEM((CHUNK + 16,), jnp.int32),
               pltpu.VMEM((CHUNK, HID), jnp.float32),
               pltpu.VMEM((CHUNK, HID), jnp.float32)] * 2
            + [pltpu.VMEM((RANGE + 1, HID), jnp.float32)]
            + [pltpu.SemaphoreType.DMA] * 4
        ),
    )
    return f(P, Q, R, src, dst)


def kernel(x, edge_index, edge_attr, smiles, batch, W1, b1, W2, b2, W3, b3, W4, b4):
    src, dst = edge_index[0], edge_index[1]
    Q1 = _edge_q(edge_attr, W1[D:])
    Q2 = _edge_q(edge_attr, W2[HID:])

    h = jnp.zeros((NPAD, D), jnp.float32).at[:N].set(x)
    for layer in range(1 + DEPTH):
        if layer == 0:
            P, R = _node_pr(h, W1[:D], b1)
            Q = Q1
        else:
            P, R = _node_pr(h, W2[:HID], b2)
            Q = Q2
        h = _sc_conv(P, Q, R, src, dst)

    batch_padded = jnp.full((NPAD,), G, jnp.int32).at[:N].set(batch)
    pooled = _pool(batch_padded, h)
    return _readout(pooled, W3, b3, W4, b4)
